# Initial kernel scaffold; baseline (speedup 1.0000x reference)
import jax
import jax.numpy as jnp
from jax.experimental import pallas as pl


def kernel(x, edge_index, W1, b1, W2, b2):
    raise NotImplementedError("write your pallas kernel here")


# speedup vs baseline: 20.0426x; 20.0426x over previous
import functools

import jax
import jax.numpy as jnp
from jax import lax
from jax.experimental import pallas as pl
from jax.experimental.pallas import tpu as pltpu
from jax.experimental.pallas import tpu_sc as plsc

N = 10000
E = 320000
F_IN = 128
HID = 128
OUT = 65
D2 = 80
NP = 10240
NC = 2
NS = 16
NW = NC * NS
EPW = E // NW
K = 40
NCH = EPW // K
ROWS_PER_TILE = NP // NS
ZR = 128
BN = 512
GRID = NP // BN

@functools.cache
def _mesh():
    return plsc.VectorSubcoreMesh(core_axis_name="c", subcore_axis_name="s",
                                  num_cores=NC, num_subcores=NS)


DEGW = 10


def _deg_body(dst_hbm, out_hbm, dst_v, ones_v, zbuf, acc, sem):
    c = lax.axis_index("c")
    s = lax.axis_index("s")
    wid = s * NC + c

    def _zfill(i, _):
        zbuf[pl.ds(i * 16, 16)] = jnp.zeros((16,), jnp.float32)
        return 0
    lax.fori_loop(0, ROWS_PER_TILE // 16, _zfill, 0)

    def _ofill(i, _):
        ones_v[pl.ds(i * 16, 16)] = jnp.ones((16,), jnp.float32)
        return 0
    lax.fori_loop(0, 48 // 16, _ofill, 0)

    pltpu.sync_copy(zbuf, acc.at[pl.ds(s * ROWS_PER_TILE, ROWS_PER_TILE)])
    plsc.subcore_barrier()

    pltpu.sync_copy(dst_hbm.at[wid], dst_v)
    ones = ones_v.at[pl.ds(0, K)]

    def _fire(jj, _):
        for b in range(DEGW):
            pltpu.async_copy(ones, acc.at[dst_v.at[jj * DEGW + b]], sem,
                             add=True)
        for b in range(DEGW):
            pltpu.make_async_copy(ones, acc.at[dst_v.at[jj * DEGW + b]],
                                  sem).wait()
        return 0
    lax.fori_loop(0, NCH // DEGW, _fire, 0)

    plsc.subcore_barrier()
    pltpu.sync_copy(acc.at[pl.ds(s * ROWS_PER_TILE, ROWS_PER_TILE)],
                    out_hbm.at[c, 0, pl.ds(s * ROWS_PER_TILE, ROWS_PER_TILE)])


@jax.jit
def _deg_call(dst3d):
    return pl.kernel(
        _deg_body,
        out_type=jax.ShapeDtypeStruct((NC, 1, NP), jnp.float32),
        mesh=_mesh(),
        scratch_types=[
            pltpu.VMEM((NCH, K), jnp.int32),
            pltpu.VMEM((48,), jnp.float32),
            pltpu.VMEM((ROWS_PER_TILE,), jnp.float32),
            pltpu.VMEM_SHARED((NP,), jnp.float32),
            pltpu.SemaphoreType.DMA,
        ],
    )(dst3d)


def _agg_body(d, g_hbm, src_hbm, dst_hbm, out_hbm,
              src_v, dst_v, buf0, buf1, zbuf, acc, sem0, sem1):
    c = lax.axis_index("c")
    s = lax.axis_index("s")
    wid = s * NC + c

    def _zero(i, _):
        for j in range(d // 16):
            zbuf[i, pl.ds(j * 16, 16)] = jnp.zeros((16,), jnp.float32)
        return 0
    lax.fori_loop(0, ZR, _zero, 0)
    for k in range(ROWS_PER_TILE // ZR):
        pltpu.sync_copy(zbuf, acc.at[pl.ds(s * ROWS_PER_TILE + k * ZR, ZR)])
    plsc.subcore_barrier()

    pltpu.sync_copy(src_hbm.at[wid], src_v)
    pltpu.sync_copy(dst_hbm.at[wid], dst_v)

    def _gather_start(j, buf, sem):
        pltpu.async_copy(g_hbm.at[src_v.at[j]], buf, sem)

    def _gather_wait(j, buf, sem):
        pltpu.make_async_copy(g_hbm.at[src_v.at[j]], buf, sem).wait()

    _gather_start(0, buf0, sem0)

    def _step(jj, _):
        j0 = jj * 2
        j1 = j0 + 1
        _gather_start(j1, buf1, sem1)
        _gather_wait(j0, buf0, sem0)
        pltpu.sync_copy(buf0, acc.at[dst_v.at[j0]], add=True)

        @pl.when(j1 + 1 < NCH)
        def _():
            _gather_start(j1 + 1, buf0, sem0)
        _gather_wait(j1, buf1, sem1)
        pltpu.sync_copy(buf1, acc.at[dst_v.at[j1]], add=True)
        return 0
    lax.fori_loop(0, NCH // 2, _step, 0)

    plsc.subcore_barrier()
    pltpu.sync_copy(acc.at[pl.ds(s * ROWS_PER_TILE, ROWS_PER_TILE)],
                    out_hbm.at[c, pl.ds(s * ROWS_PER_TILE, ROWS_PER_TILE)])


def _make_agg(d):
    @jax.jit
    def agg(g, src2d, dst2d):
        return pl.kernel(
            functools.partial(_agg_body, d),
            out_type=jax.ShapeDtypeStruct((NC, NP, d), jnp.float32),
            mesh=_mesh(),
            scratch_types=[
                pltpu.VMEM((NCH, K), jnp.int32),
                pltpu.VMEM((NCH, K), jnp.int32),
                pltpu.VMEM((K, d), jnp.float32),
                pltpu.VMEM((K, d), jnp.float32),
                pltpu.VMEM((ZR, d), jnp.float32),
                pltpu.VMEM_SHARED((NP, d), jnp.float32),
                pltpu.SemaphoreType.DMA,
                pltpu.SemaphoreType.DMA,
            ],
            compiler_params=pltpu.CompilerParams(use_tc_tiling_on_sc=False),
        )(g, src2d, dst2d)
    return agg


_agg64 = _make_agg(HID // 2)
_agg2 = _make_agg(D2)


def _tc1_body(x_ref, w1_ref, degt_ref, g1_ref, dinv_ref):
    h = jnp.dot(x_ref[...], w1_ref[...], preferred_element_type=jnp.float32)
    deg = jnp.sum(degt_ref[...], axis=1, keepdims=True) + 1.0
    dinv = lax.rsqrt(deg)
    dinv_ref[...] = dinv
    g1_ref[...] = h * dinv


@jax.jit
def _tc1(x_p, W1, degp_t):
    return pl.pallas_call(
        _tc1_body,
        grid=(GRID,),
        in_specs=[
            pl.BlockSpec((BN, F_IN), lambda i: (i, 0)),
            pl.BlockSpec((F_IN, HID), lambda i: (0, 0)),
            pl.BlockSpec((BN, NC), lambda i: (i, 0)),
        ],
        out_specs=[
            pl.BlockSpec((BN, HID), lambda i: (i, 0)),
            pl.BlockSpec((BN, 1), lambda i: (i, 0)),
        ],
        out_shape=[
            jax.ShapeDtypeStruct((NP, HID), jnp.float32),
            jax.ShapeDtypeStruct((NP, 1), jnp.float32),
        ],
    )(x_p, W1, degp_t)


def _tc2_body(accl_ref, accr_ref, g1_ref, dinv_ref, b1_ref, w2_ref, g2_ref):
    a = jnp.concatenate([accl_ref[0] + accl_ref[1],
                         accr_ref[0] + accr_ref[1]], axis=1)
    dinv = dinv_ref[...]
    h1 = jnp.maximum(dinv * (a + g1_ref[...]) + b1_ref[...], 0.0)
    g2_ref[...] = jnp.dot(h1, w2_ref[...],
                          preferred_element_type=jnp.float32) * dinv


@jax.jit
def _tc2(acc1l, acc1r, g1, dinv, b1r, W2p):
    return pl.pallas_call(
        _tc2_body,
        grid=(GRID,),
        in_specs=[
            pl.BlockSpec((NC, BN, HID // 2), lambda i: (0, i, 0)),
            pl.BlockSpec((NC, BN, HID // 2), lambda i: (0, i, 0)),
            pl.BlockSpec((BN, HID), lambda i: (i, 0)),
            pl.BlockSpec((BN, 1), lambda i: (i, 0)),
            pl.BlockSpec((1, HID), lambda i: (0, 0)),
            pl.BlockSpec((HID, D2), lambda i: (0, 0)),
        ],
        out_specs=pl.BlockSpec((BN, D2), lambda i: (i, 0)),
        out_shape=jax.ShapeDtypeStruct((NP, D2), jnp.float32),
    )(acc1l, acc1r, g1, dinv, b1r, W2p)


def _tc3_body(acc_ref, g2_ref, dinv_ref, b2_ref, out_ref):
    a = acc_ref[0] + acc_ref[1]
    z = dinv_ref[...] * (a + g2_ref[...]) + b2_ref[...]
    m = jnp.max(z, axis=1, keepdims=True)
    lse = jnp.log(jnp.sum(jnp.exp(z - m), axis=1, keepdims=True)) + m
    out_ref[...] = z - lse


@jax.jit
def _tc3(acc2, g2, dinv, b2p):
    return pl.pallas_call(
        _tc3_body,
        grid=(GRID,),
        in_specs=[
            pl.BlockSpec((NC, BN, D2), lambda i: (0, i, 0)),
            pl.BlockSpec((BN, D2), lambda i: (i, 0)),
            pl.BlockSpec((BN, 1), lambda i: (i, 0)),
            pl.BlockSpec((1, D2), lambda i: (0, 0)),
        ],
        out_specs=pl.BlockSpec((BN, D2), lambda i: (i, 0)),
        out_shape=jax.ShapeDtypeStruct((NP, D2), jnp.float32),
    )(acc2, g2, dinv, b2p)


def kernel(x, edge_index, W1, b1, W2, b2):
    src = edge_index[0]
    dst = edge_index[1]
    src2d = src.reshape(NW, NCH, K)
    dst2d = dst.reshape(NW, NCH, K)
    x_p = jnp.pad(x, ((0, NP - N), (0, 0)))
    W2p = jnp.pad(W2, ((0, 0), (0, D2 - OUT)))
    b1r = b1.reshape(1, HID)
    b2p = jnp.concatenate(
        [b2, jnp.full((D2 - OUT,), -1e30, jnp.float32)]).reshape(1, D2)

    degp = _deg_call(dst2d)
    g1, dinv = _tc1(x_p, W1, degp.reshape(NC, NP).T)
    acc1l = _agg64(g1[:, :HID // 2], src2d, dst2d)
    acc1r = _agg64(g1[:, HID // 2:], src2d, dst2d)
    g2 = _tc2(acc1l, acc1r, g1, dinv, b1r, W2p)
    acc2 = _agg2(g2, src2d, dst2d)
    out80 = _tc3(acc2, g2, dinv, b2p)
    return out80[:N, :OUT]


# speedup vs baseline: 20.3525x; 1.0155x over previous
import functools

import jax
import jax.numpy as jnp
from jax import lax
from jax.experimental import pallas as pl
from jax.experimental.pallas import tpu as pltpu
from jax.experimental.pallas import tpu_sc as plsc

N = 10000
E = 320000
F_IN = 128
HID = 128
OUT = 65
D2 = 96
NP = 10240
NC = 2
NS = 16
NW = NC * NS
K = 80
EPT = E // NS
NCH = EPT // K
KD = 40
EPW = E // NW
NCHD = EPW // KD
ROWS_PER_TILE = NP // NS
ZR = 128
BN = 512
GRID = NP // BN

_sc_params = pltpu.CompilerParams(use_tc_tiling_on_sc=False)


@functools.cache
def _mesh():
    return plsc.VectorSubcoreMesh(core_axis_name="c", subcore_axis_name="s",
                                  num_cores=NC, num_subcores=NS)


DEGW = 10


def _deg_body(dst_hbm, out_hbm, dst_v, ones_v, zbuf, acc, sem):
    c = lax.axis_index("c")
    s = lax.axis_index("s")
    wid = s * NC + c

    def _zfill(i, _):
        zbuf[pl.ds(i * 16, 16)] = jnp.zeros((16,), jnp.float32)
        return 0
    lax.fori_loop(0, ROWS_PER_TILE // 16, _zfill, 0)

    def _ofill(i, _):
        ones_v[pl.ds(i * 16, 16)] = jnp.ones((16,), jnp.float32)
        return 0
    lax.fori_loop(0, 48 // 16, _ofill, 0)

    pltpu.sync_copy(zbuf, acc.at[pl.ds(s * ROWS_PER_TILE, ROWS_PER_TILE)])
    plsc.subcore_barrier()

    pltpu.sync_copy(dst_hbm.at[wid], dst_v)
    ones = ones_v.at[pl.ds(0, KD)]

    def _fire(jj, _):
        for b in range(DEGW):
            pltpu.async_copy(ones, acc.at[dst_v.at[jj * DEGW + b]], sem,
                             add=True)
        for b in range(DEGW):
            pltpu.make_async_copy(ones, acc.at[dst_v.at[jj * DEGW + b]],
                                  sem).wait()
        return 0
    lax.fori_loop(0, NCHD // DEGW, _fire, 0)

    plsc.subcore_barrier()
    pltpu.sync_copy(acc.at[pl.ds(s * ROWS_PER_TILE, ROWS_PER_TILE)],
                    out_hbm.at[c, 0, pl.ds(s * ROWS_PER_TILE, ROWS_PER_TILE)])


@jax.jit
def _deg_call(dst3d):
    return pl.kernel(
        _deg_body,
        out_type=jax.ShapeDtypeStruct((NC, 1, NP), jnp.float32),
        mesh=_mesh(),
        scratch_types=[
            pltpu.VMEM((NCHD, KD), jnp.int32),
            pltpu.VMEM((48,), jnp.float32),
            pltpu.VMEM((ROWS_PER_TILE,), jnp.float32),
            pltpu.VMEM_SHARED((NP,), jnp.float32),
            pltpu.SemaphoreType.DMA,
        ],
    )(dst3d)


def _agg_pipeline(g_hbm, src_v, dst_v, buf0, buf1, acc, sem0, sem1):
    def _gather_start(j, buf, sem):
        pltpu.async_copy(g_hbm.at[src_v.at[j]], buf, sem)

    def _gather_wait(j, buf, sem):
        pltpu.make_async_copy(g_hbm.at[src_v.at[j]], buf, sem).wait()

    _gather_start(0, buf0, sem0)

    def _step(jj, _):
        j0 = jj * 2
        j1 = j0 + 1
        _gather_start(j1, buf1, sem1)
        _gather_wait(j0, buf0, sem0)
        pltpu.sync_copy(buf0, acc.at[dst_v.at[j0]], add=True)

        @pl.when(j1 + 1 < NCH)
        def _():
            _gather_start(j1 + 1, buf0, sem0)
        _gather_wait(j1, buf1, sem1)
        pltpu.sync_copy(buf1, acc.at[dst_v.at[j1]], add=True)
        return 0
    lax.fori_loop(0, NCH // 2, _step, 0)


def _agg_body(dh, gl_hbm, gr_hbm, src_hbm, dst_hbm, out_hbm,
              src_v, dst_v, buf0, buf1, zbuf, acc, sem0, sem1):
    c = lax.axis_index("c")
    s = lax.axis_index("s")

    def _zero(i, _):
        for j in range(dh // 16):
            zbuf[i, pl.ds(j * 16, 16)] = jnp.zeros((16,), jnp.float32)
        return 0
    lax.fori_loop(0, ZR, _zero, 0)
    for k in range(ROWS_PER_TILE // ZR):
        pltpu.sync_copy(zbuf, acc.at[pl.ds(s * ROWS_PER_TILE + k * ZR, ZR)])
    plsc.subcore_barrier()

    pltpu.sync_copy(src_hbm.at[s], src_v)
    pltpu.sync_copy(dst_hbm.at[s], dst_v)

    @pl.when(c == 0)
    def _():
        _agg_pipeline(gl_hbm, src_v, dst_v, buf0, buf1, acc, sem0, sem1)

    @pl.when(c == 1)
    def _():
        _agg_pipeline(gr_hbm, src_v, dst_v, buf0, buf1, acc, sem0, sem1)

    plsc.subcore_barrier()
    pltpu.sync_copy(acc.at[pl.ds(s * ROWS_PER_TILE, ROWS_PER_TILE)],
                    out_hbm.at[pl.ds(s * ROWS_PER_TILE, ROWS_PER_TILE), c])


def _make_agg(dh):
    @jax.jit
    def agg(gl, gr, src3d, dst3d):
        return pl.kernel(
            functools.partial(_agg_body, dh),
            out_type=jax.ShapeDtypeStruct((NP, NC, dh), jnp.float32),
            mesh=_mesh(),
            scratch_types=[
                pltpu.VMEM((NCH, K), jnp.int32),
                pltpu.VMEM((NCH, K), jnp.int32),
                pltpu.VMEM((K, dh), jnp.float32),
                pltpu.VMEM((K, dh), jnp.float32),
                pltpu.VMEM((ZR, dh), jnp.float32),
                pltpu.VMEM_SHARED((NP, dh), jnp.float32),
                pltpu.SemaphoreType.DMA,
                pltpu.SemaphoreType.DMA,
            ],
            compiler_params=_sc_params,
        )(gl, gr, src3d, dst3d)
    return agg


_agg1 = _make_agg(HID // 2)
_agg2 = _make_agg(D2 // 2)


def _tc1_body(x_ref, w1_ref, degt_ref, gl_ref, gr_ref, dinv_ref):
    h = jnp.dot(x_ref[...], w1_ref[...], preferred_element_type=jnp.float32,
                precision=lax.Precision.HIGHEST)
    deg = jnp.sum(degt_ref[...], axis=1, keepdims=True) + 1.0
    dinv = lax.rsqrt(deg)
    dinv_ref[...] = dinv
    g = h * dinv
    gl_ref[...] = g[:, :HID // 2]
    gr_ref[...] = g[:, HID // 2:]


@jax.jit
def _tc1(x_p, W1, degp_t):
    return pl.pallas_call(
        _tc1_body,
        grid=(GRID,),
        in_specs=[
            pl.BlockSpec((BN, F_IN), lambda i: (i, 0)),
            pl.BlockSpec((F_IN, HID), lambda i: (0, 0)),
            pl.BlockSpec((BN, NC), lambda i: (i, 0)),
        ],
        out_specs=[
            pl.BlockSpec((BN, HID // 2), lambda i: (i, 0)),
            pl.BlockSpec((BN, HID // 2), lambda i: (i, 0)),
            pl.BlockSpec((BN, 1), lambda i: (i, 0)),
        ],
        out_shape=[
            jax.ShapeDtypeStruct((NP, HID // 2), jnp.float32),
            jax.ShapeDtypeStruct((NP, HID // 2), jnp.float32),
            jax.ShapeDtypeStruct((NP, 1), jnp.float32),
        ],
    )(x_p, W1, degp_t)


def _tc2_body(acc_ref, gl_ref, gr_ref, dinv_ref, b1_ref, w2_ref,
              g2l_ref, g2r_ref):
    g1 = jnp.concatenate([gl_ref[...], gr_ref[...]], axis=1)
    dinv = dinv_ref[...]
    h1 = jnp.maximum(dinv * (acc_ref[...] + g1) + b1_ref[...], 0.0)
    g2 = jnp.dot(h1, w2_ref[...], preferred_element_type=jnp.float32,
                 precision=lax.Precision.HIGHEST) * dinv
    g2l_ref[...] = g2[:, :D2 // 2]
    g2r_ref[...] = g2[:, D2 // 2:]


@jax.jit
def _tc2(acc1, gl, gr, dinv, b1r, W2p):
    return pl.pallas_call(
        _tc2_body,
        grid=(GRID,),
        in_specs=[
            pl.BlockSpec((BN, HID), lambda i: (i, 0)),
            pl.BlockSpec((BN, HID // 2), lambda i: (i, 0)),
            pl.BlockSpec((BN, HID // 2), lambda i: (i, 0)),
            pl.BlockSpec((BN, 1), lambda i: (i, 0)),
            pl.BlockSpec((1, HID), lambda i: (0, 0)),
            pl.BlockSpec((HID, D2), lambda i: (0, 0)),
        ],
        out_specs=[
            pl.BlockSpec((BN, D2 // 2), lambda i: (i, 0)),
            pl.BlockSpec((BN, D2 // 2), lambda i: (i, 0)),
        ],
        out_shape=[
            jax.ShapeDtypeStruct((NP, D2 // 2), jnp.float32),
            jax.ShapeDtypeStruct((NP, D2 // 2), jnp.float32),
        ],
    )(acc1, gl, gr, dinv, b1r, W2p)


def _tc3_body(acc_ref, g2l_ref, g2r_ref, dinv_ref, b2_ref, out_ref):
    g2 = jnp.concatenate([g2l_ref[...], g2r_ref[...]], axis=1)
    z = dinv_ref[...] * (acc_ref[...] + g2) + b2_ref[...]
    m = jnp.max(z, axis=1, keepdims=True)
    lse = jnp.log(jnp.sum(jnp.exp(z - m), axis=1, keepdims=True)) + m
    out_ref[...] = z - lse


@jax.jit
def _tc3(acc2, g2l, g2r, dinv, b2p):
    return pl.pallas_call(
        _tc3_body,
        grid=(GRID,),
        in_specs=[
            pl.BlockSpec((BN, D2), lambda i: (i, 0)),
            pl.BlockSpec((BN, D2 // 2), lambda i: (i, 0)),
            pl.BlockSpec((BN, D2 // 2), lambda i: (i, 0)),
            pl.BlockSpec((BN, 1), lambda i: (i, 0)),
            pl.BlockSpec((1, D2), lambda i: (0, 0)),
        ],
        out_specs=pl.BlockSpec((BN, D2), lambda i: (i, 0)),
        out_shape=jax.ShapeDtypeStruct((NP, D2), jnp.float32),
    )(acc2, g2l, g2r, dinv, b2p)


def kernel(x, edge_index, W1, b1, W2, b2):
    src = edge_index[0]
    dst = edge_index[1]
    src3d = src.reshape(NS, NCH, K)
    dst3d = dst.reshape(NS, NCH, K)
    dst3d_deg = dst.reshape(NW, NCHD, KD)
    x_p = jnp.pad(x, ((0, NP - N), (0, 0)))
    W2p = jnp.pad(W2, ((0, 0), (0, D2 - OUT)))
    b1r = b1.reshape(1, HID)
    b2p = jnp.concatenate(
        [b2, jnp.full((D2 - OUT,), -1e30, jnp.float32)]).reshape(1, D2)

    degp = _deg_call(dst3d_deg)
    gl, gr, dinv = _tc1(x_p, W1, degp.reshape(NC, NP).T)
    acc1 = _agg1(gl, gr, src3d, dst3d)
    g2l, g2r = _tc2(acc1.reshape(NP, HID), gl, gr, dinv, b1r, W2p)
    acc2 = _agg2(g2l, g2r, src3d, dst3d)
    out80 = _tc3(acc2.reshape(NP, D2), g2l, g2r, dinv, b2p)
    return out80[:N, :OUT]
